# Initial kernel scaffold; baseline (speedup 1.0000x reference)
import jax
import jax.numpy as jnp
from jax.experimental import pallas as pl


def kernel(x, edge_index, batch, W1, b1, W2, b2, W3, b3, Wf1, bf1, Wf2, bf2, eps):
    raise NotImplementedError("write your pallas kernel here")


# speedup vs baseline: 1.8875x; 1.8875x over previous
import functools

import jax
import jax.numpy as jnp
from jax.experimental import pallas as pl

N = 16000
E = 512000
B = 32
IN_DIM = 128
H1 = 64
H2 = 32
K = 7
DEC_OUT = 124750
DEC_PAD = 124928


def _dec_block(h_ref, w_ref, b_ref, o_ref):
    h = h_ref[...]
    w = w_ref[...]
    b = b_ref[...]
    o_ref[...] = jax.nn.sigmoid(
        jax.lax.dot_general(h, w, (((1,), (0,)), ((), ())),
                            preferred_element_type=jnp.float32) + b)


@jax.jit
def _decoder(h, Wf2p, bf2p):
    grid = DEC_PAD // 2048
    return pl.pallas_call(
        _dec_block,
        grid=(grid,),
        in_specs=[
            pl.BlockSpec((B, H1), lambda i: (0, 0)),
            pl.BlockSpec((H1, 2048), lambda i: (0, i)),
            pl.BlockSpec((1, 2048), lambda i: (0, i)),
        ],
        out_specs=pl.BlockSpec((B, 2048), lambda i: (0, i)),
        out_shape=jax.ShapeDtypeStruct((B, DEC_PAD), jnp.float32),
    )(h, Wf2p, bf2p)


def kernel(x, edge_index, batch, W1, b1, W2, b2, W3, b3, Wf1, bf1, Wf2, bf2, eps):
    row, col = edge_index[0], edge_index[1]
    deg = jax.ops.segment_sum(jnp.ones((E,), jnp.float32), row, num_segments=N)
    dis = jnp.where(deg > 0, jax.lax.rsqrt(jnp.maximum(deg, 1e-12)), 0.0)

    def aprop(z):
        return jax.ops.segment_sum(jnp.take(z, col, axis=0), row, num_segments=N)

    def aprop_t(z):
        return jax.ops.segment_sum(jnp.take(z, row, axis=0), col, num_segments=N)

    def sprop(z):
        return -dis[:, None] * aprop(dis[:, None] * z)

    def spropt(z):
        return -dis[:, None] * aprop_t(dis[:, None] * z)

    C = jnp.einsum('nd,kdh->knh', x, W1)
    bk1 = C[K - 1]
    bk2 = jnp.zeros_like(bk1)
    for k in range(K - 2, 0, -1):
        bk1, bk2 = C[k] + 2.0 * sprop(bk1) - bk2, bk1
    out1 = C[0] + sprop(bk1) - bk2 + b1
    h1 = jax.nn.relu(out1)

    Y0 = (batch[:, None] == jnp.arange(B)[None, :]).astype(jnp.float32)
    Y1 = spropt(Y0)
    pooled = [Y0.T @ h1, Y1.T @ h1]
    Ya, Yb = Y0, Y1
    for k in range(2, K):
        Ya, Yb = Yb, 2.0 * spropt(Yb) - Ya
        pooled.append(Yb.T @ h1)
    P = jnp.stack(pooled)

    counts = jax.ops.segment_sum(jnp.ones((N,), jnp.float32), batch, num_segments=B)
    denom = jnp.maximum(counts, 1.0)[:, None]
    mu = (jnp.einsum('kbh,kho->bo', P, W2) + counts[:, None] * b2) / denom
    logvar = (jnp.einsum('kbh,kho->bo', P, W3) + counts[:, None] * b3) / denom

    z = mu + eps * jnp.exp(0.5 * logvar)
    h = jax.nn.relu(z @ Wf1 + bf1)

    Wf2p = jnp.pad(Wf2, ((0, 0), (0, DEC_PAD - DEC_OUT)))
    bf2p = jnp.pad(bf2, (0, DEC_PAD - DEC_OUT)).reshape(1, DEC_PAD)
    adj = _decoder(h, Wf2p, bf2p)[:, :DEC_OUT]
    return (adj, mu, logvar)
